# baseline (speedup 1.0000x reference)
import functools

import jax
import jax.numpy as jnp
from jax import lax
from jax.experimental import pallas as pl
from jax.experimental.pallas import tpu as pltpu
from jax.experimental.pallas import tpu_sc as plsc

B = 4096
F = 26
D_DENSE = 13
FEAT = 4000
DIM = 16
TOTAL = F * FEAT
FF = F * F

NC = 2
NS = 16
NW = NC * NS
NB = B // NW
CHUNK = 4
NCHUNK = NB // CHUNK
PAIR_ROWS = CHUNK * FF
SUB = CHUNK * F
NSUB = PAIR_ROWS // SUB

WBT = 1000


def _pad_weights_body(w_ref, o_ref):
    w = w_ref[0, 0, :].reshape(WBT, 1)
    o_ref[...] = jnp.concatenate(
        [w, jnp.zeros((WBT, DIM - 1), jnp.float32)], axis=1)


def _pad_weights(weight_sparse):
    return pl.pallas_call(
        _pad_weights_body,
        grid=(TOTAL // WBT,),
        in_specs=[pl.BlockSpec((1, 1, WBT), lambda t: (t, 0, 0))],
        out_specs=pl.BlockSpec((WBT, DIM), lambda t: (t, 0)),
        out_shape=jax.ShapeDtypeStruct((TOTAL, DIM), jnp.float32),
    )(weight_sparse.reshape(TOTAL // WBT, 1, WBT))


def _sc_gather_cross(flat_emb, w16, pair_idx, sp_flat):
    mesh = plsc.VectorSubcoreMesh(core_axis_name="c", subcore_axis_name="s")

    @functools.partial(
        pl.kernel,
        mesh=mesh,
        out_type=jax.ShapeDtypeStruct((B, DIM), jnp.float32),
        scratch_types=[
            pltpu.VMEM((PAIR_ROWS,), jnp.int32),
            pltpu.VMEM((NB * F,), jnp.int32),
            pltpu.VMEM((PAIR_ROWS, DIM), jnp.float32),
            pltpu.VMEM((SUB, DIM), jnp.float32),
            pltpu.VMEM((NB, DIM), jnp.float32),
            pltpu.SemaphoreType.DMA,
        ],
        compiler_params=pltpu.CompilerParams(use_tc_tiling_on_sc=False),
    )
    def k(emb_hbm, w16_hbm, pidx_hbm, sp_hbm, out_hbm,
          pidx_v, sidx_v, rows_v, wrows_v, out_v, sem):
        wid = lax.axis_index("s") * NC + lax.axis_index("c")
        pltpu.sync_copy(sp_hbm.at[pl.ds(wid * (NB * F), NB * F)], sidx_v)

        def chunk_body(c, carry):
            pltpu.sync_copy(
                pidx_hbm.at[pl.ds(wid * (NB * FF) + c * PAIR_ROWS,
                                  PAIR_ROWS)], pidx_v)
            copies = []
            for s in range(NSUB):
                copies.append(pltpu.async_copy(
                    emb_hbm.at[pidx_v.at[pl.ds(s * SUB, SUB)]],
                    rows_v.at[pl.ds(s * SUB, SUB)], sem))
            copies.append(pltpu.async_copy(
                w16_hbm.at[sidx_v.at[pl.ds(c * SUB, SUB)]], wrows_v, sem))
            for cp in copies:
                cp.wait()

            def b_body(bb, carry2):
                r0 = bb * FF
                w0 = bb * F
                acc = jnp.zeros((DIM,), jnp.float32)
                for i in range(F - 1):
                    for j in range(i + 1, F):
                        acc = acc + (rows_v[r0 + i * F + j, :] *
                                     rows_v[r0 + j * F + i, :])
                for i in range(F):
                    acc = acc + wrows_v[w0 + i, :]
                out_v[c * CHUNK + bb, :] = acc
                return carry2

            lax.fori_loop(0, CHUNK, b_body, 0, unroll=False)
            return carry

        lax.fori_loop(0, NCHUNK, chunk_body, 0, unroll=False)
        pltpu.sync_copy(out_v, out_hbm.at[pl.ds(wid * NB, NB)])

    return k(flat_emb, w16, pair_idx, sp_flat)


def _final_body(dense_ref, wd_ref, b_ref, part_ref, o_ref):
    lin = jnp.sum(dense_ref[...] * wd_ref[...], axis=1, keepdims=True)
    cross = jnp.sum(part_ref[...], axis=1, keepdims=True)
    o_ref[...] = jax.nn.sigmoid(lin + cross + b_ref[0, 0])


def _final(dense, wd_row, bias11, partial):
    return pl.pallas_call(
        _final_body,
        out_shape=jax.ShapeDtypeStruct((B, 1), jnp.float32),
    )(dense, wd_row, bias11, partial)


def kernel(dense_input, sparse_input, bias, weight_dense, weight_sparse,
           embed_tables):
    offs = jnp.arange(F, dtype=jnp.int32) * FEAT
    sp = sparse_input + offs[None, :]
    sp_flat = sp.reshape(B * F)
    tab_off = jnp.arange(F, dtype=jnp.int32) * TOTAL
    pair_idx = (sp[:, :, None] + tab_off[None, None, :]).reshape(B * FF)
    flat_emb = embed_tables.reshape(F * TOTAL, DIM)
    w16 = _pad_weights(weight_sparse)
    partial = _sc_gather_cross(flat_emb, w16, pair_idx, sp_flat)
    return _final(dense_input, weight_dense.reshape(1, D_DENSE),
                  bias.reshape(1, 1), partial)
